# baseline (speedup 1.0000x reference)
import functools
import jax
import jax.numpy as jnp
from jax import lax
from jax.experimental import pallas as pl
from jax.experimental.pallas import tpu as pltpu
from jax.experimental.pallas import tpu_sc as plsc

_K = 64
_N = 32768
_ROWS = 128
_L = 16
_NV = _N // _L
_NBUCKET = 512
_CAP = 4096
_NC = 2
_NS = 16
_NW = _NC * _NS
_RPW = _ROWS // _NW


def _skey(v):
    s = lax.bitcast_convert_type(v, jnp.int32)
    return s ^ (lax.shift_right_arithmetic(s, 31) & jnp.int32(0x7FFFFFFF))


def _sc_body(x_hbm, o_hbm, row_v, hist_v, tot_v, candk_v, candi_v):
    wid = lax.axis_index("s") * _NC + lax.axis_index("c")
    lane = lax.broadcasted_iota(jnp.int32, (_L,), 0)
    zero_v = jnp.zeros((_L,), jnp.int32)
    ones_v = jnp.ones((_L,), jnp.int32)
    lane_off = lane * _NBUCKET

    for rr in range(_RPW):
        row = wid * _RPW + rr
        pltpu.sync_copy(x_hbm.at[row], row_v)

        def zh(i, _):
            hist_v[pl.ds(pl.multiple_of(i * _L, 8), _L)] = zero_v
            return 0
        lax.fori_loop(0, _NBUCKET * _L // _L, zh, 0)

        def hstep(i, _):
            v = row_v[pl.ds(pl.multiple_of(i * _L, 8), _L)]
            b = lax.shift_right_arithmetic(_skey(v), 23) + jnp.int32(256)
            plsc.addupdate_scatter(hist_v, [lane_off + b], ones_v)
            return 0
        lax.fori_loop(0, _NV, hstep, 0)

        def rstep(j, _):
            base = pl.multiple_of(j * _L, 8)
            def cstep(c, a):
                return a + hist_v[pl.ds(pl.multiple_of(c * _NBUCKET, 8) + base, _L)]
            tot_v[pl.ds(base, _L)] = lax.fori_loop(0, _L, cstep, zero_v)
            return 0
        lax.fori_loop(0, _NBUCKET // _L, rstep, 0)

        def sstep(jj, carry):
            above, bestv = carry
            j = _NBUCKET // _L - 1 - jj
            base = j * _L
            v = tot_v[pl.ds(pl.multiple_of(base, 8), _L)]
            rc = lax.rev(jnp.cumsum(lax.rev(v, (0,))), (0,)) + above
            ids = base + lane
            bestv = jnp.maximum(bestv, jnp.where(rc >= _K, ids, -1))
            return (jnp.max(rc), bestv)
        _, bestv = lax.fori_loop(
            0, _NBUCKET // _L, sstep,
            (jnp.int32(0), jnp.full((_L,), -1, jnp.int32)))
        bstar = jnp.max(bestv)

        def tstep(j, carry):
            a_tot, a_hi = carry
            v = tot_v[pl.ds(pl.multiple_of(j * _L, 8), _L)]
            ids = j * _L + lane
            return (a_tot + jnp.where(ids >= bstar, v, 0),
                    a_hi + jnp.where(ids > bstar, v, 0))
        a_tot, _ = lax.fori_loop(0, _NBUCKET // _L, tstep, (zero_v, zero_v))
        c_tot = jnp.sum(a_tot)

        k0 = lax.shift_left(bstar - jnp.int32(256), 23)
        def comp(i, carry):
            offv, idxv = carry
            o = pl.multiple_of(i * _L, 8)
            v = row_v[pl.ds(o, _L)]
            key = _skey(v)
            b = lax.shift_right_arithmetic(key, 23) + jnp.int32(256)
            m = b >= bstar
            dk = jnp.where(b > bstar, jnp.int32(0x800000),
                           key & jnp.int32(0x7FFFFF))
            ci = jnp.cumsum(m.astype(jnp.int32))
            pos = offv + ci - 1
            okm = m & (pos < _CAP)
            plsc.store_scatter(candk_v, [pos], dk, mask=okm)
            plsc.store_scatter(candi_v, [pos], idxv, mask=okm)
            return (offv + jnp.sum(m.astype(jnp.int32)), idxv + _L)
        lax.fori_loop(0, _NV, comp, (zero_v, lane))

        c_eff = jnp.minimum(c_tot, jnp.int32(_CAP))
        tpos = c_eff + lane
        plsc.store_scatter(candk_v, [jnp.where(tpos < _CAP, tpos, _CAP - 1)],
                           jnp.full((_L,), -1, jnp.int32), mask=tpos < _CAP)
        nv = (c_eff + jnp.int32(_L - 1)) // _L

        def bit_step(b_, t):
            b = jnp.int32(22) - b_
            cand = t | lax.shift_left(jnp.int32(1), b)
            def cnt(j, a):
                ck = candk_v[pl.ds(pl.multiple_of(j * _L, 8), _L)]
                return a + (ck >= cand).astype(jnp.int32)
            c = jnp.sum(lax.fori_loop(0, nv, cnt, zero_v))
            return jnp.where(c >= _K, cand, t)
        t_low = lax.fori_loop(0, 23, bit_step, jnp.int32(0))

        def cnt2(j, carry):
            a_ge, a_gt = carry
            ck = candk_v[pl.ds(pl.multiple_of(j * _L, 8), _L)]
            return (a_ge + (ck >= t_low).astype(jnp.int32),
                    a_gt + (ck > t_low).astype(jnp.int32))
        a_ge, a_gt = lax.fori_loop(0, nv, cnt2, (zero_v, zero_v))
        n_ge = jnp.sum(a_ge)
        r = jnp.int32(_K) - jnp.sum(a_gt)

        def tie():
            def jb(b_, J):
                b = jnp.int32(14) - b_
                cand = J | lax.shift_left(jnp.int32(1), b)
                def cnt(j, a):
                    ck = candk_v[pl.ds(pl.multiple_of(j * _L, 8), _L)]
                    civ = candi_v[pl.ds(pl.multiple_of(j * _L, 8), _L)]
                    m = (ck == t_low) & (civ < cand)
                    return a + m.astype(jnp.int32)
                c = jnp.sum(lax.fori_loop(0, nv, cnt, zero_v))
                return jnp.where(c < r, cand, J)
            return lax.fori_loop(0, 15, jb, jnp.int32(0))
        J = lax.cond(n_ge == _K, lambda: jnp.int32(_N - 1), tie)

        t_key = k0 + t_low
        def ap(i, idxv):
            o = pl.multiple_of(i * _L, 8)
            v = row_v[pl.ds(o, _L)]
            key = _skey(v)
            m = (key > t_key) | ((key == t_key) & (idxv <= J))
            row_v[pl.ds(o, _L)] = jnp.where(m, v, 0.0)
            return idxv + _L
        lax.fori_loop(0, _NV, ap, lane)
        pltpu.sync_copy(row_v, o_hbm.at[row])


def kernel(x):
    mesh = plsc.VectorSubcoreMesh(core_axis_name="c", subcore_axis_name="s")
    fn = functools.partial(
        pl.kernel,
        mesh=mesh,
        compiler_params=pltpu.CompilerParams(needs_layout_passes=False),
        out_type=jax.ShapeDtypeStruct((_ROWS, _N), jnp.float32),
        scratch_types=[
            pltpu.VMEM((_N,), jnp.float32),
            pltpu.VMEM((_NBUCKET * _L,), jnp.int32),
            pltpu.VMEM((_NBUCKET,), jnp.int32),
            pltpu.VMEM((_CAP,), jnp.int32),
            pltpu.VMEM((_CAP,), jnp.int32),
        ],
    )(_sc_body)
    return fn(x)


# speedup vs baseline: 2.7376x; 2.7376x over previous
import functools
import jax
import jax.numpy as jnp
from jax import lax
from jax.experimental import pallas as pl
from jax.experimental.pallas import tpu as pltpu
from jax.experimental.pallas import tpu_sc as plsc

_K = 64
_N = 32768
_ROWS = 128
_L = 16
_NV = _N // _L
_NBUCKET = 256
_CAP = 4096
_NC = 2
_NS = 16
_NW = _NC * _NS
_RPW = _ROWS // _NW
_NBUF = 3


def _skey(v):
    s = lax.bitcast_convert_type(v, jnp.int32)
    return s ^ (lax.shift_right_arithmetic(s, 31) & jnp.int32(0x7FFFFFFF))


def _sc_body(x_hbm, o_hbm, b0_v, b1_v, b2_v, hist_v, tot_v, candk_v, candi_v,
             ls0, ls1, ls2, ss0, ss1, ss2):
    wid = lax.axis_index("s") * _NC + lax.axis_index("c")
    lane = lax.broadcasted_iota(jnp.int32, (_L,), 0)
    zero_v = jnp.zeros((_L,), jnp.int32)
    ones_v = jnp.ones((_L,), jnp.int32)
    lane_off = lane * _NBUCKET
    bufs = [b0_v, b1_v, b2_v]
    lsems = [ls0, ls1, ls2]
    ssems = [ss0, ss1, ss2]

    def start_load(rr):
        return pltpu.async_copy(x_hbm.at[wid * _RPW + rr], bufs[rr % _NBUF],
                                lsems[rr % _NBUF])

    ld = [None] * _RPW
    st = [None] * _RPW
    for rr in range(min(_NBUF, _RPW)):
        ld[rr] = start_load(rr)
    st_waited = set()

    for rr in range(_RPW):
        if rr >= 2 and rr + 1 < _RPW:
            st[rr - 2].wait()
            st_waited.add(rr - 2)
            ld[rr + 1] = start_load(rr + 1)
        row_v = bufs[rr % _NBUF]
        ld[rr].wait()

        @plsc.parallel_loop(0, _NBUCKET * _L // _L, unroll=8)
        def _(i):
            hist_v[pl.ds(pl.multiple_of(i * _L, 8), _L)] = zero_v

        @plsc.parallel_loop(0, _NV, unroll=8)
        def _(i):
            v = row_v[pl.ds(pl.multiple_of(i * _L, 8), _L)]
            b = lax.shift_right_arithmetic(_skey(v), 24) + jnp.int32(128)
            plsc.addupdate_scatter(hist_v, [lane_off + b], ones_v)

        @plsc.parallel_loop(0, _NBUCKET // _L, unroll=2)
        def _(j):
            base = pl.multiple_of(j * _L, 8)
            acc = hist_v[pl.ds(base, _L)]
            for c in range(1, _L):
                acc = acc + hist_v[pl.ds(base + c * _NBUCKET, _L)]
            tot_v[pl.ds(base, _L)] = acc

        def sstep(jj, carry):
            above, bestv = carry
            j = _NBUCKET // _L - 1 - jj
            base = j * _L
            v = tot_v[pl.ds(pl.multiple_of(base, 8), _L)]
            rc = lax.rev(jnp.cumsum(lax.rev(v, (0,))), (0,)) + above
            ids = base + lane
            bestv = jnp.maximum(bestv, jnp.where(rc >= _K, ids, -1))
            return (jnp.max(rc), bestv)
        _, bestv = lax.fori_loop(
            0, _NBUCKET // _L, sstep,
            (jnp.int32(0), jnp.full((_L,), -1, jnp.int32)))
        bstar = jnp.max(bestv)

        def tstep(j, a_tot):
            v = tot_v[pl.ds(pl.multiple_of(j * _L, 8), _L)]
            ids = j * _L + lane
            return a_tot + jnp.where(ids >= bstar, v, 0)
        a_tot = lax.fori_loop(0, _NBUCKET // _L, tstep, zero_v)
        c_tot = jnp.sum(a_tot)

        k0 = lax.shift_left(bstar - jnp.int32(128), 24)

        @plsc.parallel_loop(0, _NV, unroll=4, carry=(zero_v, lane))
        def comp_out(i, carry):
            offv, idxv = carry
            v = row_v[pl.ds(pl.multiple_of(i * _L, 8), _L)]
            key = _skey(v)
            b = lax.shift_right_arithmetic(key, 24) + jnp.int32(128)
            m = b >= bstar
            dk = jnp.where(b > bstar, jnp.int32(0x1000000),
                           key & jnp.int32(0xFFFFFF))
            ci = jnp.cumsum(m.astype(jnp.int32))
            pos = jnp.minimum(offv + ci - 1, jnp.int32(_CAP - 1))
            okm = m & (pos < _CAP)
            plsc.store_scatter(candk_v, [pos], dk, mask=okm)
            plsc.store_scatter(candi_v, [pos], idxv, mask=okm)
            return (offv + plsc.all_reduce_population_count(m), idxv + _L)

        c_eff = jnp.minimum(c_tot, jnp.int32(_CAP))
        tpos = c_eff + lane
        plsc.store_scatter(candk_v, [jnp.minimum(tpos, jnp.int32(_CAP - 1))],
                           jnp.full((_L,), -1, jnp.int32), mask=tpos < _CAP)
        nv = (c_eff + jnp.int32(_L - 1)) // _L

        def bit_step(b_, t):
            b = jnp.int32(23) - b_
            cand = t | lax.shift_left(jnp.int32(1), b)
            def cnt(j, a):
                ck = candk_v[pl.ds(pl.multiple_of(j * _L, 8), _L)]
                return a + (ck >= cand).astype(jnp.int32)
            c = jnp.sum(lax.fori_loop(0, nv, cnt, zero_v))
            return jnp.where(c >= _K, cand, t)
        t_low = lax.fori_loop(0, 24, bit_step, jnp.int32(0))

        def cnt2(j, carry):
            a_ge, a_gt = carry
            ck = candk_v[pl.ds(pl.multiple_of(j * _L, 8), _L)]
            return (a_ge + (ck >= t_low).astype(jnp.int32),
                    a_gt + (ck > t_low).astype(jnp.int32))
        a_ge, a_gt = lax.fori_loop(0, nv, cnt2, (zero_v, zero_v))
        n_ge = jnp.sum(a_ge)
        r = jnp.int32(_K) - jnp.sum(a_gt)

        def tie():
            def jb(b_, J):
                b = jnp.int32(14) - b_
                cand = J | lax.shift_left(jnp.int32(1), b)
                def cnt(j, a):
                    ck = candk_v[pl.ds(pl.multiple_of(j * _L, 8), _L)]
                    civ = candi_v[pl.ds(pl.multiple_of(j * _L, 8), _L)]
                    m = (ck == t_low) & (civ < cand)
                    return a + m.astype(jnp.int32)
                c = jnp.sum(lax.fori_loop(0, nv, cnt, zero_v))
                return jnp.where(c < r, cand, J)
            return lax.fori_loop(0, 15, jb, jnp.int32(0))
        J = lax.cond(n_ge == _K, lambda: jnp.int32(_N - 1), tie)

        t_key = k0 + t_low

        @plsc.parallel_loop(0, _NV, unroll=8, carry=lane)
        def ap_out(i, idxv):
            o = pl.multiple_of(i * _L, 8)
            v = row_v[pl.ds(o, _L)]
            key = _skey(v)
            m = (key > t_key) | ((key == t_key) & (idxv <= J))
            row_v[pl.ds(o, _L)] = jnp.where(m, v, 0.0)
            return idxv + _L

        st[rr] = pltpu.async_copy(row_v, o_hbm.at[wid * _RPW + rr],
                                  ssems[rr % _NBUF])

    for rr in range(_RPW):
        if rr not in st_waited:
            st[rr].wait()


def kernel(x):
    mesh = plsc.VectorSubcoreMesh(core_axis_name="c", subcore_axis_name="s")
    fn = functools.partial(
        pl.kernel,
        mesh=mesh,
        compiler_params=pltpu.CompilerParams(needs_layout_passes=False),
        out_type=jax.ShapeDtypeStruct((_ROWS, _N), jnp.float32),
        scratch_types=[
            pltpu.VMEM((_N,), jnp.float32),
            pltpu.VMEM((_N,), jnp.float32),
            pltpu.VMEM((_N,), jnp.float32),
            pltpu.VMEM((_NBUCKET * _L,), jnp.int32),
            pltpu.VMEM((_NBUCKET,), jnp.int32),
            pltpu.VMEM((_CAP,), jnp.int32),
            pltpu.VMEM((_CAP,), jnp.int32),
            pltpu.SemaphoreType.DMA,
            pltpu.SemaphoreType.DMA,
            pltpu.SemaphoreType.DMA,
            pltpu.SemaphoreType.DMA,
            pltpu.SemaphoreType.DMA,
            pltpu.SemaphoreType.DMA,
        ],
    )(_sc_body)
    return fn(x)


# speedup vs baseline: 3.5278x; 1.2887x over previous
import functools
import jax
import jax.numpy as jnp
from jax import lax
from jax.experimental import pallas as pl
from jax.experimental.pallas import tpu as pltpu
from jax.experimental.pallas import tpu_sc as plsc

_K = 64
_N = 32768
_ROWS = 128
_L = 16
_NV = _N // _L
_NBUCKET = 256
_CAP = 4096
_NC = 2
_NS = 16
_NW = _NC * _NS
_RPW = _ROWS // _NW
_NBUF = 3


def _skey(v):
    s = lax.bitcast_convert_type(v, jnp.int32)
    return s ^ (lax.shift_right_arithmetic(s, 31) & jnp.int32(0x7FFFFFFF))


def _sc_body(x_hbm, o_hbm, b0_v, b1_v, b2_v, hist_v, tot_v, candk_v, candi_v,
             cand2k_v, cand2i_v, ls0, ls1, ls2, ss0, ss1, ss2):
    wid = lax.axis_index("s") * _NC + lax.axis_index("c")
    lane = lax.broadcasted_iota(jnp.int32, (_L,), 0)
    zero_v = jnp.zeros((_L,), jnp.int32)
    ones_v = jnp.ones((_L,), jnp.int32)
    lane_off = lane * _NBUCKET
    bufs = [b0_v, b1_v, b2_v]
    lsems = [ls0, ls1, ls2]
    ssems = [ss0, ss1, ss2]

    def start_load(rr):
        return pltpu.async_copy(x_hbm.at[wid * _RPW + rr], bufs[rr % _NBUF],
                                lsems[rr % _NBUF])

    ld = [None] * _RPW
    st = [None] * _RPW
    for rr in range(min(_NBUF, _RPW)):
        ld[rr] = start_load(rr)
    st_waited = set()

    for rr in range(_RPW):
        if rr >= 2 and rr + 1 < _RPW:
            st[rr - 2].wait()
            st_waited.add(rr - 2)
            ld[rr + 1] = start_load(rr + 1)
        row_v = bufs[rr % _NBUF]
        ld[rr].wait()

        @plsc.parallel_loop(0, _NBUCKET * _L // _L, unroll=8)
        def _(i):
            hist_v[pl.ds(pl.multiple_of(i * _L, 8), _L)] = zero_v

        @plsc.parallel_loop(0, _NV, unroll=8)
        def _(i):
            v = row_v[pl.ds(pl.multiple_of(i * _L, 8), _L)]
            b = lax.shift_right_arithmetic(_skey(v), 24) + jnp.int32(128)
            plsc.addupdate_scatter(hist_v, [lane_off + b], ones_v)

        @plsc.parallel_loop(0, _NBUCKET // _L, unroll=2)
        def _(j):
            base = pl.multiple_of(j * _L, 8)
            acc = hist_v[pl.ds(base, _L)]
            for c in range(1, _L):
                acc = acc + hist_v[pl.ds(base + c * _NBUCKET, _L)]
            tot_v[pl.ds(base, _L)] = acc

        def sstep(jj, carry):
            above, bestv = carry
            j = _NBUCKET // _L - 1 - jj
            base = j * _L
            v = tot_v[pl.ds(pl.multiple_of(base, 8), _L)]
            rc = lax.rev(jnp.cumsum(lax.rev(v, (0,))), (0,)) + above
            ids = base + lane
            bestv = jnp.maximum(bestv, jnp.where(rc >= _K, ids, -1))
            return (jnp.max(rc), bestv)
        _, bestv = lax.fori_loop(
            0, _NBUCKET // _L, sstep,
            (jnp.int32(0), jnp.full((_L,), -1, jnp.int32)))
        bstar = jnp.max(bestv)

        def tstep(j, a_tot):
            v = tot_v[pl.ds(pl.multiple_of(j * _L, 8), _L)]
            ids = j * _L + lane
            return a_tot + jnp.where(ids >= bstar, v, 0)
        a_tot = lax.fori_loop(0, _NBUCKET // _L, tstep, zero_v)
        c_tot = jnp.sum(a_tot)

        k0 = lax.shift_left(bstar - jnp.int32(128), 24)

        @plsc.parallel_loop(0, _NV, unroll=4, carry=(jnp.full((_L,), -1, jnp.int32), lane))
        def comp_out(i, carry):
            offm1, idxv = carry
            v = row_v[pl.ds(pl.multiple_of(i * _L, 8), _L)]
            key = _skey(v)
            b = lax.shift_right_arithmetic(key, 24) + jnp.int32(128)
            m = b >= bstar
            dk = jnp.where(b > bstar, jnp.int32(0x1000000),
                           key & jnp.int32(0xFFFFFF))
            ci = jnp.cumsum(m.astype(jnp.int32))
            pos = offm1 + ci
            okm = m & (pos < _CAP)
            plsc.store_scatter(candk_v, [pos], dk, mask=okm)
            plsc.store_scatter(candi_v, [pos], idxv, mask=okm)
            return (offm1 + plsc.all_reduce_population_count(m), idxv + _L)

        c_eff = jnp.minimum(c_tot, jnp.int32(_CAP))
        tpos = c_eff + lane
        plsc.store_scatter(candk_v, [jnp.minimum(tpos, jnp.int32(_CAP - 1))],
                           jnp.full((_L,), -1, jnp.int32), mask=tpos < _CAP)
        nv = (c_eff + jnp.int32(_L - 1)) // _L

        def bit_hi(b_, t):
            b = jnp.int32(23) - b_
            cand = t | lax.shift_left(jnp.int32(1), b)

            @plsc.parallel_loop(0, nv, unroll=4, carry=zero_v)
            def acc_out(j, a):
                ck = candk_v[pl.ds(pl.multiple_of(j * _L, 8), _L)]
                return a + (ck >= cand).astype(jnp.int32)
            c = jnp.sum(acc_out)
            return jnp.where(c >= _K, cand, t)
        t_hi = lax.fori_loop(0, 8, bit_hi, jnp.int32(0))

        @plsc.parallel_loop(0, nv, unroll=2, carry=jnp.full((_L,), -1, jnp.int32))
        def rc_out(j, offm1):
            o = pl.multiple_of(j * _L, 8)
            ck = candk_v[pl.ds(o, _L)]
            civ = candi_v[pl.ds(o, _L)]
            m = ck >= t_hi
            ci = jnp.cumsum(m.astype(jnp.int32))
            pos = offm1 + ci
            okm = m & (pos < _CAP)
            plsc.store_scatter(cand2k_v, [pos], ck, mask=okm)
            plsc.store_scatter(cand2i_v, [pos], civ, mask=okm)
            return offm1 + plsc.all_reduce_population_count(m)
        c2 = jnp.max(rc_out) + jnp.int32(1)
        tpos2 = c2 + lane
        plsc.store_scatter(cand2k_v, [jnp.minimum(tpos2, jnp.int32(_CAP - 1))],
                           jnp.full((_L,), -1, jnp.int32), mask=tpos2 < _CAP)
        nv2 = (c2 + jnp.int32(_L - 1)) // _L

        def bit_lo(b_, t):
            b = jnp.int32(15) - b_
            cand = t | lax.shift_left(jnp.int32(1), b)

            @plsc.parallel_loop(0, nv2, unroll=2, carry=zero_v)
            def acc_out(j, a):
                ck = cand2k_v[pl.ds(pl.multiple_of(j * _L, 8), _L)]
                return a + (ck >= cand).astype(jnp.int32)
            c = jnp.sum(acc_out)
            return jnp.where(c >= _K, cand, t)
        t_low = lax.fori_loop(0, 16, bit_lo, t_hi)

        def cnt2(j, carry):
            a_ge, a_gt = carry
            ck = cand2k_v[pl.ds(pl.multiple_of(j * _L, 8), _L)]
            return (a_ge + (ck >= t_low).astype(jnp.int32),
                    a_gt + (ck > t_low).astype(jnp.int32))
        a_ge, a_gt = lax.fori_loop(0, nv2, cnt2, (zero_v, zero_v))
        n_ge = jnp.sum(a_ge)
        r = jnp.int32(_K) - jnp.sum(a_gt)
        t_key = k0 + t_low

        def ap_fast(_):
            tb = jnp.where(t_key >= 0, t_key, t_key ^ jnp.int32(0x7FFFFFFF))
            t_f = lax.bitcast_convert_type(tb, jnp.float32)

            @plsc.parallel_loop(0, _NV, unroll=8)
            def _(i):
                o = pl.multiple_of(i * _L, 8)
                v = row_v[pl.ds(o, _L)]
                row_v[pl.ds(o, _L)] = jnp.where(v >= t_f, v, 0.0)
            return jnp.int32(0)

        def ap_tie(_):
            def jb(b_, J):
                b = jnp.int32(14) - b_
                cand = J | lax.shift_left(jnp.int32(1), b)
                def cnt(j, a):
                    ck = cand2k_v[pl.ds(pl.multiple_of(j * _L, 8), _L)]
                    civ = cand2i_v[pl.ds(pl.multiple_of(j * _L, 8), _L)]
                    m = (ck == t_low) & (civ < cand)
                    return a + m.astype(jnp.int32)
                c = jnp.sum(lax.fori_loop(0, nv2, cnt, zero_v))
                return jnp.where(c < r, cand, J)
            J = lax.fori_loop(0, 15, jb, jnp.int32(0))

            @plsc.parallel_loop(0, _NV, unroll=8, carry=lane)
            def ap_out(i, idxv):
                o = pl.multiple_of(i * _L, 8)
                v = row_v[pl.ds(o, _L)]
                key = _skey(v)
                m = (key > t_key) | ((key == t_key) & (idxv <= J))
                row_v[pl.ds(o, _L)] = jnp.where(m, v, 0.0)
                return idxv + _L
            return jnp.int32(0)

        lax.cond(n_ge == _K, ap_fast, ap_tie, jnp.int32(0))

        st[rr] = pltpu.async_copy(row_v, o_hbm.at[wid * _RPW + rr],
                                  ssems[rr % _NBUF])

    for rr in range(_RPW):
        if rr not in st_waited:
            st[rr].wait()


def kernel(x):
    mesh = plsc.VectorSubcoreMesh(core_axis_name="c", subcore_axis_name="s")
    fn = functools.partial(
        pl.kernel,
        mesh=mesh,
        compiler_params=pltpu.CompilerParams(needs_layout_passes=False),
        out_type=jax.ShapeDtypeStruct((_ROWS, _N), jnp.float32),
        scratch_types=[
            pltpu.VMEM((_N,), jnp.float32),
            pltpu.VMEM((_N,), jnp.float32),
            pltpu.VMEM((_N,), jnp.float32),
            pltpu.VMEM((_NBUCKET * _L,), jnp.int32),
            pltpu.VMEM((_NBUCKET,), jnp.int32),
            pltpu.VMEM((_CAP,), jnp.int32),
            pltpu.VMEM((_CAP,), jnp.int32),
            pltpu.VMEM((_CAP,), jnp.int32),
            pltpu.VMEM((_CAP,), jnp.int32),
            pltpu.SemaphoreType.DMA,
            pltpu.SemaphoreType.DMA,
            pltpu.SemaphoreType.DMA,
            pltpu.SemaphoreType.DMA,
            pltpu.SemaphoreType.DMA,
            pltpu.SemaphoreType.DMA,
        ],
    )(_sc_body)
    return fn(x)


# speedup vs baseline: 3.6365x; 1.0308x over previous
import functools
import jax
import jax.numpy as jnp
from jax import lax
from jax.experimental import pallas as pl
from jax.experimental.pallas import tpu as pltpu
from jax.experimental.pallas import tpu_sc as plsc

_K = 64
_N = 32768
_ROWS = 128
_L = 16
_NV = _N // _L
_NBUCKET = 256
_CAP = 4096
_CAP2 = 1024
_NC = 2
_NS = 16
_NW = _NC * _NS
_RPW = _ROWS // _NW


def _skey(v):
    s = lax.bitcast_convert_type(v, jnp.int32)
    return s ^ (lax.shift_right_arithmetic(s, 31) & jnp.int32(0x7FFFFFFF))


def _sc_body(x_hbm, o_hbm, b0_v, b1_v, out_v, hist_v, tot_v,
             candk_v, candi_v, candv_v, c2k_v, c2i_v, c2v_v,
             ki0_v, ki1_v, ls0, ls1, osem):
    wid = lax.axis_index("s") * _NC + lax.axis_index("c")
    lane = lax.broadcasted_iota(jnp.int32, (_L,), 0)
    zero_v = jnp.zeros((_L,), jnp.int32)
    ones_v = jnp.ones((_L,), jnp.int32)
    fzero_v = jnp.zeros((_L,), jnp.float32)
    neg1_v = jnp.full((_L,), -1, jnp.int32)
    lane_off = lane * _NBUCKET
    bufs = [b0_v, b1_v]
    lsems = [ls0, ls1]
    kbufs = [ki0_v, ki1_v]

    @plsc.parallel_loop(0, _NV, unroll=8)
    def _(i):
        out_v[pl.ds(pl.multiple_of(i * _L, 8), _L)] = fzero_v

    def start_load(rr):
        return pltpu.async_copy(x_hbm.at[wid * _RPW + rr], bufs[rr % 2],
                                lsems[rr % 2])

    ld = [None] * _RPW
    for rr in range(min(2, _RPW)):
        ld[rr] = start_load(rr)
    odma = [None] * _RPW

    def hist_reduce_scan():
        @plsc.parallel_loop(0, _NBUCKET // _L, unroll=2)
        def _(j):
            base = pl.multiple_of(j * _L, 8)
            acc = hist_v[pl.ds(base, _L)]
            for c in range(1, _L):
                acc = acc + hist_v[pl.ds(base + c * _NBUCKET, _L)]
            tot_v[pl.ds(base, _L)] = acc

        def sstep(jj, carry):
            above, bestv = carry
            j = _NBUCKET // _L - 1 - jj
            base = j * _L
            v = tot_v[pl.ds(pl.multiple_of(base, 8), _L)]
            rc = lax.rev(jnp.cumsum(lax.rev(v, (0,))), (0,)) + above
            ids = base + lane
            bestv = jnp.maximum(bestv, jnp.where(rc >= _K, ids, -1))
            return (jnp.max(rc), bestv)
        _, bestv = lax.fori_loop(0, _NBUCKET // _L, sstep,
                                 (jnp.int32(0), neg1_v))
        return jnp.max(bestv)

    def zero_hist():
        @plsc.parallel_loop(0, _NBUCKET, unroll=8)
        def _(i):
            hist_v[pl.ds(pl.multiple_of(i * _L, 8), _L)] = zero_v

    for rr in range(_RPW):
        row_v = bufs[rr % 2]
        ld[rr].wait()
        zero_hist()

        @plsc.parallel_loop(0, _NV, unroll=8)
        def _(i):
            v = row_v[pl.ds(pl.multiple_of(i * _L, 8), _L)]
            b = lax.shift_right_arithmetic(_skey(v), 24) + jnp.int32(128)
            plsc.addupdate_scatter(hist_v, [lane_off + b], ones_v)

        bstar = hist_reduce_scan()

        def tstep(j, a_tot):
            v = tot_v[pl.ds(pl.multiple_of(j * _L, 8), _L)]
            ids = j * _L + lane
            return a_tot + jnp.where(ids >= bstar, v, 0)
        a_tot = lax.fori_loop(0, _NBUCKET // _L, tstep, zero_v)
        c_tot = jnp.sum(a_tot)

        k0 = lax.shift_left(bstar - jnp.int32(128), 24)

        @plsc.parallel_loop(0, _NV, unroll=4, carry=(neg1_v, lane))
        def comp_out(i, carry):
            offm1, idxv = carry
            v = row_v[pl.ds(pl.multiple_of(i * _L, 8), _L)]
            key = _skey(v)
            b = lax.shift_right_arithmetic(key, 24) + jnp.int32(128)
            m = b >= bstar
            dk = jnp.where(b > bstar, jnp.int32(0x1000000),
                           key & jnp.int32(0xFFFFFF))
            ci = jnp.cumsum(m.astype(jnp.int32))
            pos = offm1 + ci
            okm = m & (pos < _CAP)
            plsc.store_scatter(candk_v, [pos], dk, mask=okm)
            plsc.store_scatter(candi_v, [pos], idxv, mask=okm)
            plsc.store_scatter(candv_v, [pos], v, mask=okm)
            return (offm1 + plsc.all_reduce_population_count(m), idxv + _L)

        if rr + 2 < _RPW:
            ld[rr + 2] = start_load(rr + 2)

        c_eff = jnp.minimum(c_tot, jnp.int32(_CAP))
        tpos = c_eff + lane
        plsc.store_scatter(candk_v, [jnp.minimum(tpos, jnp.int32(_CAP - 1))],
                           neg1_v, mask=tpos < _CAP)
        nv = (c_eff + jnp.int32(_L - 1)) // _L

        zero_hist()

        @plsc.parallel_loop(0, nv, unroll=4)
        def _(j):
            dk = candk_v[pl.ds(pl.multiple_of(j * _L, 8), _L)]
            b2 = jnp.clip(lax.shift_right_arithmetic(dk, 16),
                          jnp.int32(0), jnp.int32(255))
            plsc.addupdate_scatter(hist_v, [lane_off + b2], ones_v)

        t_hi = lax.shift_left(hist_reduce_scan(), 16)

        @plsc.parallel_loop(0, nv, unroll=2, carry=neg1_v)
        def rc_out(j, offm1):
            o = pl.multiple_of(j * _L, 8)
            ck = candk_v[pl.ds(o, _L)]
            m = ck >= t_hi
            ci = jnp.cumsum(m.astype(jnp.int32))
            pos = offm1 + ci
            okm = m & (pos < _CAP2)
            plsc.store_scatter(c2k_v, [pos], ck, mask=okm)
            plsc.store_scatter(c2i_v, [pos], candi_v[pl.ds(o, _L)], mask=okm)
            plsc.store_scatter(c2v_v, [pos], candv_v[pl.ds(o, _L)], mask=okm)
            return offm1 + plsc.all_reduce_population_count(m)
        c2 = jnp.max(rc_out) + jnp.int32(1)
        tpos2 = c2 + lane
        plsc.store_scatter(c2k_v, [jnp.minimum(tpos2, jnp.int32(_CAP2 - 1))],
                           neg1_v, mask=tpos2 < _CAP2)
        nv2 = (jnp.minimum(c2, jnp.int32(_CAP2)) + jnp.int32(_L - 1)) // _L

        def bit_lo(b_, t):
            b = jnp.int32(15) - b_
            cand = t | lax.shift_left(jnp.int32(1), b)

            @plsc.parallel_loop(0, nv2, unroll=2, carry=zero_v)
            def acc_out(j, a):
                ck = c2k_v[pl.ds(pl.multiple_of(j * _L, 8), _L)]
                return a + (ck >= cand).astype(jnp.int32)
            c = jnp.sum(acc_out)
            return jnp.where(c >= _K, cand, t)
        t_low = lax.fori_loop(0, 16, bit_lo, t_hi)

        def cnt2(j, carry):
            a_ge, a_gt = carry
            ck = c2k_v[pl.ds(pl.multiple_of(j * _L, 8), _L)]
            return (a_ge + (ck >= t_low).astype(jnp.int32),
                    a_gt + (ck > t_low).astype(jnp.int32))
        a_ge, a_gt = lax.fori_loop(0, nv2, cnt2, (zero_v, zero_v))
        n_ge = jnp.sum(a_ge)
        r = jnp.int32(_K) - jnp.sum(a_gt)

        def tie(_):
            def jb(b_, J):
                b = jnp.int32(14) - b_
                cand = J | lax.shift_left(jnp.int32(1), b)
                def cnt(j, a):
                    ck = c2k_v[pl.ds(pl.multiple_of(j * _L, 8), _L)]
                    civ = c2i_v[pl.ds(pl.multiple_of(j * _L, 8), _L)]
                    m = (ck == t_low) & (civ < cand)
                    return a + m.astype(jnp.int32)
                c = jnp.sum(lax.fori_loop(0, nv2, cnt, zero_v))
                return jnp.where(c < r, cand, J)
            return lax.fori_loop(0, 15, jb, jnp.int32(0))
        J = lax.cond(n_ge == _K, lambda _: jnp.int32(_N - 1), tie,
                     jnp.int32(0))

        if rr >= 1:
            odma[rr - 1].wait()
            kprev = kbufs[(rr - 1) % 2]
            for j in range(_K // _L):
                zi = kprev[pl.ds(j * _L, _L)]
                plsc.store_scatter(out_v, [zi], fzero_v)

        kcur = kbufs[rr % 2]

        @plsc.parallel_loop(0, nv2, unroll=2, carry=neg1_v)
        def kc_out(j, offm1):
            o = pl.multiple_of(j * _L, 8)
            ck = c2k_v[pl.ds(o, _L)]
            civ = c2i_v[pl.ds(o, _L)]
            cv = c2v_v[pl.ds(o, _L)]
            m = (ck > t_low) | ((ck == t_low) & (civ <= J))
            ci = jnp.cumsum(m.astype(jnp.int32))
            pos = offm1 + ci
            okm = m & (pos < _K)
            plsc.store_scatter(out_v, [civ], cv, mask=okm)
            plsc.store_scatter(kcur, [pos], civ, mask=okm)
            return offm1 + plsc.all_reduce_population_count(m)

        odma[rr] = pltpu.async_copy(out_v, o_hbm.at[wid * _RPW + rr], osem)

    odma[_RPW - 1].wait()


def kernel(x):
    mesh = plsc.VectorSubcoreMesh(core_axis_name="c", subcore_axis_name="s")
    fn = functools.partial(
        pl.kernel,
        mesh=mesh,
        compiler_params=pltpu.CompilerParams(needs_layout_passes=False),
        out_type=jax.ShapeDtypeStruct((_ROWS, _N), jnp.float32),
        scratch_types=[
            pltpu.VMEM((_N,), jnp.float32),
            pltpu.VMEM((_N,), jnp.float32),
            pltpu.VMEM((_N,), jnp.float32),
            pltpu.VMEM((_NBUCKET * _L,), jnp.int32),
            pltpu.VMEM((_NBUCKET,), jnp.int32),
            pltpu.VMEM((_CAP,), jnp.int32),
            pltpu.VMEM((_CAP,), jnp.int32),
            pltpu.VMEM((_CAP,), jnp.float32),
            pltpu.VMEM((_CAP2,), jnp.int32),
            pltpu.VMEM((_CAP2,), jnp.int32),
            pltpu.VMEM((_CAP2,), jnp.float32),
            pltpu.VMEM((_K,), jnp.int32),
            pltpu.VMEM((_K,), jnp.int32),
            pltpu.SemaphoreType.DMA,
            pltpu.SemaphoreType.DMA,
            pltpu.SemaphoreType.DMA,
        ],
    )(_sc_body)
    return fn(x)


# speedup vs baseline: 5.9857x; 1.6460x over previous
import functools
import jax
import jax.numpy as jnp
from jax import lax
from jax.experimental import pallas as pl
from jax.experimental.pallas import tpu as pltpu
from jax.experimental.pallas import tpu_sc as plsc

_K = 64
_N = 32768
_ROWS = 128
_L = 16
_NV = _N // _L
_CAP = 4096
_NC = 2
_NS = 16
_NW = _NC * _NS
_RPW = _ROWS // _NW
_GUESS0 = 0x40000000
_MARGIN = 1 << 21
_IMIN = -0x80000000


def _skey(v):
    s = lax.bitcast_convert_type(v, jnp.int32)
    return s ^ (lax.shift_right_arithmetic(s, 31) & jnp.int32(0x7FFFFFFF))


def _sc_body(x_hbm, o_hbm, b0_v, b1_v, out_v, candk_v, candi_v,
             ki0_v, ki1_v, ls0, ls1, osem):
    wid = lax.axis_index("s") * _NC + lax.axis_index("c")
    lane = lax.broadcasted_iota(jnp.int32, (_L,), 0)
    zero_v = jnp.zeros((_L,), jnp.int32)
    fzero_v = jnp.zeros((_L,), jnp.float32)
    neg1_v = jnp.full((_L,), -1, jnp.int32)
    bufs = [b0_v, b1_v]
    lsems = [ls0, ls1]
    kbufs = [ki0_v, ki1_v]

    @plsc.parallel_loop(0, _NV, unroll=8)
    def _(i):
        out_v[pl.ds(pl.multiple_of(i * _L, 8), _L)] = fzero_v

    def start_load(rr):
        return pltpu.async_copy(x_hbm.at[wid * _RPW + rr], bufs[rr % 2],
                                lsems[rr % 2])

    ld = [None] * _RPW
    for rr in range(min(2, _RPW)):
        ld[rr] = start_load(rr)
    odma = [None] * _RPW

    t_spec = jnp.int32(_GUESS0)
    for rr in range(_RPW):
        row_v = bufs[rr % 2]
        ld[rr].wait()

        def not_ok(carry):
            _, _, _, c, fin = carry
            bad = (c < _K) | (c > _CAP)
            return bad & jnp.logical_not(fin)

        def attempt(carry):
            t_prev, lo, hi, c, fin = carry
            ran = c >= 0
            lo = jnp.where(ran & (c > _CAP), t_prev, lo)
            hi = jnp.where(ran & (c < _K), t_prev, hi)
            ulo = lax.bitcast_convert_type(lo, jnp.uint32) ^ jnp.uint32(0x80000000)
            uhi = lax.bitcast_convert_type(hi, jnp.uint32) ^ jnp.uint32(0x80000000)
            umid = ulo + lax.shift_right_logical(uhi - ulo, jnp.uint32(1))
            mid = lax.bitcast_convert_type(umid ^ jnp.uint32(0x80000000), jnp.int32)
            narrow = (uhi - ulo) <= jnp.uint32(1)
            fin = ran & narrow
            t_s = jnp.where(ran, jnp.where(narrow, lo, mid), t_prev)
            tsu = lax.bitcast_convert_type(t_s, jnp.uint32)

            @plsc.parallel_loop(0, _NV, unroll=4, carry=(neg1_v, lane))
            def comp_out(i, cr):
                offm1, idxv = cr
                v = row_v[pl.ds(pl.multiple_of(i * _L, 8), _L)]
                key = _skey(v)
                m = key >= t_s
                du = lax.bitcast_convert_type(key, jnp.uint32) - tsu
                ci = jnp.cumsum(m.astype(jnp.int32))
                pos = offm1 + ci
                okm = m & (pos < _CAP)
                plsc.store_scatter(candk_v, [pos],
                                   lax.bitcast_convert_type(du, jnp.int32),
                                   mask=okm)
                plsc.store_scatter(candi_v, [pos], idxv, mask=okm)
                return (offm1 + plsc.all_reduce_population_count(m), idxv + _L)

            c_new = jnp.max(comp_out[0]) + jnp.int32(1)
            return (t_s, lo, hi, c_new, fin)

        t_s, _, _, c_tot, _ = lax.while_loop(
            not_ok, attempt,
            (t_spec, jnp.int32(_IMIN), jnp.int32(0x7FFFFFFF),
             jnp.int32(-1), jnp.bool_(False)))

        c_eff = jnp.minimum(c_tot, jnp.int32(_CAP))
        tpos = c_eff + lane
        okp = tpos < _CAP
        tposc = jnp.minimum(tpos, jnp.int32(_CAP - 1))
        plsc.store_scatter(candk_v, [tposc], zero_v, mask=okp)
        plsc.store_scatter(candi_v, [tposc], jnp.full((_L,), _N, jnp.int32),
                           mask=okp)
        nv = (c_eff + jnp.int32(_L - 1)) // _L

        def bit_step(b_, t):
            b = (jnp.uint32(31) - b_.astype(jnp.uint32))
            cand = t | lax.shift_left(jnp.uint32(1), b)

            @plsc.parallel_loop(0, nv, unroll=2, carry=zero_v)
            def acc_out(j, a):
                ck = lax.bitcast_convert_type(
                    candk_v[pl.ds(pl.multiple_of(j * _L, 8), _L)], jnp.uint32)
                return a + (ck >= cand).astype(jnp.int32)
            c = jnp.sum(acc_out)
            return jnp.where(c >= _K, cand, t)
        t_du = lax.fori_loop(0, 32, bit_step, jnp.uint32(0))

        def cnt2(j, carry):
            a_ge, a_gt = carry
            ck = lax.bitcast_convert_type(
                candk_v[pl.ds(pl.multiple_of(j * _L, 8), _L)], jnp.uint32)
            return (a_ge + (ck >= t_du).astype(jnp.int32),
                    a_gt + (ck > t_du).astype(jnp.int32))
        a_ge, a_gt = lax.fori_loop(0, nv, cnt2, (zero_v, zero_v))
        n_ge = jnp.sum(a_ge)
        r = jnp.int32(_K) - jnp.sum(a_gt)

        def tie(_):
            def jb(b_, J):
                b = jnp.int32(14) - b_
                cand = J | lax.shift_left(jnp.int32(1), b)
                def cnt(j, a):
                    o = pl.multiple_of(j * _L, 8)
                    ck = lax.bitcast_convert_type(candk_v[pl.ds(o, _L)],
                                                  jnp.uint32)
                    civ = candi_v[pl.ds(o, _L)]
                    m = (ck == t_du) & (civ < cand)
                    return a + m.astype(jnp.int32)
                c = jnp.sum(lax.fori_loop(0, nv, cnt, zero_v))
                return jnp.where(c < r, cand, J)
            return lax.fori_loop(0, 15, jb, jnp.int32(0))
        J = lax.cond(n_ge == _K, lambda _: jnp.int32(_N - 1), tie,
                     jnp.int32(0))

        if rr >= 1:
            odma[rr - 1].wait()
            kprev = kbufs[(rr - 1) % 2]
            for j in range(_K // _L):
                zi = kprev[pl.ds(j * _L, _L)]
                plsc.store_scatter(out_v, [zi], fzero_v)

        kcur = kbufs[rr % 2]

        @plsc.parallel_loop(0, nv, unroll=2, carry=neg1_v)
        def kc_out(j, offm1):
            o = pl.multiple_of(j * _L, 8)
            ck = lax.bitcast_convert_type(candk_v[pl.ds(o, _L)], jnp.uint32)
            civ = candi_v[pl.ds(o, _L)]
            m = (ck > t_du) | ((ck == t_du) & (civ <= J))
            ci = jnp.cumsum(m.astype(jnp.int32))
            pos = offm1 + ci
            okm = m & (pos < _K)
            cv = plsc.load_gather(row_v, [jnp.where(okm, civ, 0)])
            plsc.store_scatter(out_v, [civ], cv, mask=okm)
            plsc.store_scatter(kcur, [pos], civ, mask=okm)
            return offm1 + plsc.all_reduce_population_count(m)

        odma[rr] = pltpu.async_copy(out_v, o_hbm.at[wid * _RPW + rr], osem)

        t_key = lax.bitcast_convert_type(
            lax.bitcast_convert_type(t_s, jnp.uint32) + t_du, jnp.int32)
        t_spec = jnp.where(t_key < jnp.int32(_IMIN + _MARGIN),
                           jnp.int32(_IMIN), t_key - jnp.int32(_MARGIN))
        if rr + 2 < _RPW:
            ld[rr + 2] = start_load(rr + 2)

    odma[_RPW - 1].wait()


def kernel(x):
    mesh = plsc.VectorSubcoreMesh(core_axis_name="c", subcore_axis_name="s")
    fn = functools.partial(
        pl.kernel,
        mesh=mesh,
        compiler_params=pltpu.CompilerParams(needs_layout_passes=False),
        out_type=jax.ShapeDtypeStruct((_ROWS, _N), jnp.float32),
        scratch_types=[
            pltpu.VMEM((_N,), jnp.float32),
            pltpu.VMEM((_N,), jnp.float32),
            pltpu.VMEM((_N,), jnp.float32),
            pltpu.VMEM((_CAP,), jnp.int32),
            pltpu.VMEM((_CAP,), jnp.int32),
            pltpu.VMEM((_K,), jnp.int32),
            pltpu.VMEM((_K,), jnp.int32),
            pltpu.SemaphoreType.DMA,
            pltpu.SemaphoreType.DMA,
            pltpu.SemaphoreType.DMA,
        ],
    )(_sc_body)
    return fn(x)


# speedup vs baseline: 6.3134x; 1.0547x over previous
import functools
import jax
import jax.numpy as jnp
from jax import lax
from jax.experimental import pallas as pl
from jax.experimental.pallas import tpu as pltpu
from jax.experimental.pallas import tpu_sc as plsc

_K = 64
_N = 32768
_ROWS = 128
_L = 16
_NV = _N // _L
_CAP = 4096
_NC = 2
_NS = 16
_NW = _NC * _NS
_RPW = _ROWS // _NW
_GUESS0 = 0x40000000
_MARGIN = 1 << 21
_IMIN = -0x80000000


def _skey(v):
    s = lax.bitcast_convert_type(v, jnp.int32)
    return s ^ (lax.shift_right_arithmetic(s, 31) & jnp.int32(0x7FFFFFFF))


def _sc_body(x_hbm, o_hbm, b0_v, b1_v, out_v, candk_v, candi_v,
             ki0_v, ki1_v, ls0, ls1, osem):
    wid = lax.axis_index("s") * _NC + lax.axis_index("c")
    lane = lax.broadcasted_iota(jnp.int32, (_L,), 0)
    zero_v = jnp.zeros((_L,), jnp.int32)
    fzero_v = jnp.zeros((_L,), jnp.float32)
    neg1_v = jnp.full((_L,), -1, jnp.int32)
    bufs = [b0_v, b1_v]
    lsems = [ls0, ls1]
    kbufs = [ki0_v, ki1_v]

    @plsc.parallel_loop(0, _NV, unroll=8)
    def _(i):
        out_v[pl.ds(pl.multiple_of(i * _L, 8), _L)] = fzero_v

    def start_load(rr):
        return pltpu.async_copy(x_hbm.at[wid * _RPW + rr], bufs[rr % 2],
                                lsems[rr % 2])

    ld = [None] * _RPW
    for rr in range(min(2, _RPW)):
        ld[rr] = start_load(rr)
    odma = [None] * _RPW

    t_spec = jnp.int32(_GUESS0)
    for rr in range(_RPW):
        row_v = bufs[rr % 2]
        ld[rr].wait()

        def not_ok(carry):
            _, _, _, c, fin = carry
            bad = (c < _K) | (c > _CAP)
            return bad & jnp.logical_not(fin)

        def attempt(carry):
            t_prev, lo, hi, c, fin = carry
            ran = c >= 0
            lo = jnp.where(ran & (c > _CAP), t_prev, lo)
            hi = jnp.where(ran & (c < _K), t_prev, hi)
            ulo = lax.bitcast_convert_type(lo, jnp.uint32) ^ jnp.uint32(0x80000000)
            uhi = lax.bitcast_convert_type(hi, jnp.uint32) ^ jnp.uint32(0x80000000)
            umid = ulo + lax.shift_right_logical(uhi - ulo, jnp.uint32(1))
            mid = lax.bitcast_convert_type(umid ^ jnp.uint32(0x80000000), jnp.int32)
            narrow = (uhi - ulo) <= jnp.uint32(1)
            fin = ran & narrow
            t_s = jnp.where(ran, jnp.where(narrow, lo, mid), t_prev)
            tsu = lax.bitcast_convert_type(t_s, jnp.uint32)

            @plsc.parallel_loop(0, _NV, unroll=8, carry=(neg1_v, lane))
            def comp_out(i, cr):
                offm1, idxv = cr
                v = row_v[pl.ds(pl.multiple_of(i * _L, 8), _L)]
                key = _skey(v)
                m = key >= t_s
                du = lax.bitcast_convert_type(key, jnp.uint32) - tsu
                ci = jnp.cumsum(m.astype(jnp.int32))
                pos = offm1 + ci
                okm = m & (pos < _CAP)
                plsc.store_scatter(candk_v, [pos],
                                   lax.bitcast_convert_type(du, jnp.int32),
                                   mask=okm)
                plsc.store_scatter(candi_v, [pos], idxv, mask=okm)
                return (offm1 + plsc.all_reduce_population_count(m), idxv + _L)

            c_new = jnp.max(comp_out[0]) + jnp.int32(1)
            return (t_s, lo, hi, c_new, fin)

        t_s, _, _, c_tot, _ = lax.while_loop(
            not_ok, attempt,
            (t_spec, jnp.int32(_IMIN), jnp.int32(0x7FFFFFFF),
             jnp.int32(-1), jnp.bool_(False)))

        c_eff = jnp.minimum(c_tot, jnp.int32(_CAP))
        tpos = c_eff + lane
        okp = tpos < _CAP
        tposc = jnp.minimum(tpos, jnp.int32(_CAP - 1))
        plsc.store_scatter(candk_v, [tposc], zero_v, mask=okp)
        plsc.store_scatter(candi_v, [tposc], jnp.full((_L,), _N, jnp.int32),
                           mask=okp)
        nv = (c_eff + jnp.int32(_L - 1)) // _L

        @plsc.parallel_loop(0, nv, unroll=2, carry=zero_v)
        def mx_out(j, a):
            ck = lax.bitcast_convert_type(
                candk_v[pl.ds(pl.multiple_of(j * _L, 8), _L)], jnp.uint32)
            return jnp.maximum(a, lax.bitcast_convert_type(
                lax.shift_right_logical(ck, jnp.uint32(1)), jnp.int32))
        mx = jnp.max(mx_out)
        mxf = lax.bitcast_convert_type(
            lax.convert_element_type(mx, jnp.float32), jnp.int32)
        b0 = jnp.clip(lax.shift_right_logical(mxf, 23) - jnp.int32(125),
                      jnp.int32(1), jnp.int32(32))

        def bit_step(b_, t):
            b = (b0 - jnp.int32(1) - b_).astype(jnp.uint32)
            cand = t | lax.shift_left(jnp.uint32(1), b)

            @plsc.parallel_loop(0, nv, unroll=2, carry=zero_v)
            def acc_out(j, a):
                ck = lax.bitcast_convert_type(
                    candk_v[pl.ds(pl.multiple_of(j * _L, 8), _L)], jnp.uint32)
                return a + (ck >= cand).astype(jnp.int32)
            c = jnp.sum(acc_out)
            return jnp.where(c >= _K, cand, t)
        t_du = lax.fori_loop(0, b0, bit_step, jnp.uint32(0))

        def cnt2(j, carry):
            a_ge, a_gt = carry
            ck = lax.bitcast_convert_type(
                candk_v[pl.ds(pl.multiple_of(j * _L, 8), _L)], jnp.uint32)
            return (a_ge + (ck >= t_du).astype(jnp.int32),
                    a_gt + (ck > t_du).astype(jnp.int32))
        a_ge, a_gt = lax.fori_loop(0, nv, cnt2, (zero_v, zero_v))
        n_ge = jnp.sum(a_ge)
        r = jnp.int32(_K) - jnp.sum(a_gt)

        def tie(_):
            def jb(b_, J):
                b = jnp.int32(14) - b_
                cand = J | lax.shift_left(jnp.int32(1), b)
                def cnt(j, a):
                    o = pl.multiple_of(j * _L, 8)
                    ck = lax.bitcast_convert_type(candk_v[pl.ds(o, _L)],
                                                  jnp.uint32)
                    civ = candi_v[pl.ds(o, _L)]
                    m = (ck == t_du) & (civ < cand)
                    return a + m.astype(jnp.int32)
                c = jnp.sum(lax.fori_loop(0, nv, cnt, zero_v))
                return jnp.where(c < r, cand, J)
            return lax.fori_loop(0, 15, jb, jnp.int32(0))
        J = lax.cond(n_ge == _K, lambda _: jnp.int32(_N - 1), tie,
                     jnp.int32(0))

        if rr >= 1:
            odma[rr - 1].wait()
            kprev = kbufs[(rr - 1) % 2]
            for j in range(_K // _L):
                zi = kprev[pl.ds(j * _L, _L)]
                plsc.store_scatter(out_v, [zi], fzero_v)

        kcur = kbufs[rr % 2]

        @plsc.parallel_loop(0, nv, unroll=2, carry=neg1_v)
        def kc_out(j, offm1):
            o = pl.multiple_of(j * _L, 8)
            ck = lax.bitcast_convert_type(candk_v[pl.ds(o, _L)], jnp.uint32)
            civ = candi_v[pl.ds(o, _L)]
            m = (ck > t_du) | ((ck == t_du) & (civ <= J))
            ci = jnp.cumsum(m.astype(jnp.int32))
            pos = offm1 + ci
            okm = m & (pos < _K)
            cv = plsc.load_gather(row_v, [jnp.where(okm, civ, 0)])
            plsc.store_scatter(out_v, [civ], cv, mask=okm)
            plsc.store_scatter(kcur, [pos], civ, mask=okm)
            return offm1 + plsc.all_reduce_population_count(m)

        odma[rr] = pltpu.async_copy(out_v, o_hbm.at[wid * _RPW + rr], osem)

        t_key = lax.bitcast_convert_type(
            lax.bitcast_convert_type(t_s, jnp.uint32) + t_du, jnp.int32)
        t_spec = jnp.where(t_key < jnp.int32(_IMIN + _MARGIN),
                           jnp.int32(_IMIN), t_key - jnp.int32(_MARGIN))
        if rr + 2 < _RPW:
            ld[rr + 2] = start_load(rr + 2)

    odma[_RPW - 1].wait()


def kernel(x):
    mesh = plsc.VectorSubcoreMesh(core_axis_name="c", subcore_axis_name="s")
    fn = functools.partial(
        pl.kernel,
        mesh=mesh,
        compiler_params=pltpu.CompilerParams(needs_layout_passes=False),
        out_type=jax.ShapeDtypeStruct((_ROWS, _N), jnp.float32),
        scratch_types=[
            pltpu.VMEM((_N,), jnp.float32),
            pltpu.VMEM((_N,), jnp.float32),
            pltpu.VMEM((_N,), jnp.float32),
            pltpu.VMEM((_CAP,), jnp.int32),
            pltpu.VMEM((_CAP,), jnp.int32),
            pltpu.VMEM((_K,), jnp.int32),
            pltpu.VMEM((_K,), jnp.int32),
            pltpu.SemaphoreType.DMA,
            pltpu.SemaphoreType.DMA,
            pltpu.SemaphoreType.DMA,
        ],
    )(_sc_body)
    return fn(x)
